# baseline (speedup 1.0000x reference)
import functools

import jax
import jax.numpy as jnp
from jax import lax
from jax.experimental import pallas as pl
from jax.experimental.pallas import tpu as pltpu
from jax.experimental.pallas import tpu_sc as plsc

_N_NODES = 10000
_N_EDGES = 320000
_D = 128
_N_GRAPHS = 64
_N_CLASSES = 10

_NC = 2
_NS = 16
_NW = _NC * _NS
_EPW = _N_EDGES // _NW
_PITCH = 10112
_COFF = _PITCH
_ACC = 657408
_APT = _ACC // _NS
_ZB = _APT // 2


_CHUNKS = _EPW // 16
_HA = 4992
_HB = _EPW - _HA


def _sc_histogram(edge_index, batch, ones, zeros):
    mesh = plsc.VectorSubcoreMesh(
        core_axis_name="c", subcore_axis_name="s",
        num_cores=_NC, num_subcores=_NS)

    @functools.partial(
        pl.kernel,
        out_type=jax.ShapeDtypeStruct((_NC * _ACC,), jnp.float32),
        mesh=mesh,
        scratch_types=[
            pltpu.VMEM((_EPW,), jnp.int32),
            pltpu.VMEM((_EPW,), jnp.int32),
            pltpu.VMEM((_EPW,), jnp.int32),
            pltpu.VMEM((_HA,), jnp.int32),
            pltpu.VMEM((_HB,), jnp.int32),
            pltpu.VMEM((_EPW,), jnp.float32),
            pltpu.VMEM((_ZB,), jnp.float32),
            pltpu.VMEM_SHARED((_ACC,), jnp.float32),
            pltpu.SemaphoreType.DMA,
            pltpu.SemaphoreType.DMA,
            pltpu.SemaphoreType.DMA,
            pltpu.SemaphoreType.DMA,
            pltpu.SemaphoreType.DMA,
            pltpu.SemaphoreType.DMA,
            pltpu.SemaphoreType.DMA,
            pltpu.SemaphoreType.DMA,
            pltpu.SemaphoreType.DMA,
            pltpu.SemaphoreType.DMA,
        ],
    )
    def hist(edge_hbm, batch_hbm, ones_hbm, zeros_hbm, out_hbm,
             row_v, col_v, g_v, idxA, idxB, ones_v, zbuf, shared,
             s_row, s_col, s_ones, s_zero, s_sh, s_g1, s_g2, s_sc,
             s_rd, s_wr):
        cid = lax.axis_index("c")
        sid = lax.axis_index("s")
        wid = sid * _NC + cid
        base = wid * _EPW

        d_row = pltpu.async_copy(edge_hbm.at[pl.ds(base, _EPW)], row_v,
                                 s_row)
        d_col = pltpu.async_copy(edge_hbm.at[pl.ds(_N_EDGES + base, _EPW)],
                                 col_v, s_col)
        d_on = pltpu.async_copy(ones_hbm, ones_v, s_ones)
        d_z = pltpu.async_copy(zeros_hbm, zbuf, s_zero)

        d_row.wait()
        d_g1 = pltpu.async_copy(
            batch_hbm.at[row_v.at[pl.ds(0, _HA)]],
            g_v.at[pl.ds(0, _HA)], s_g1)
        d_g2 = pltpu.async_copy(
            batch_hbm.at[row_v.at[pl.ds(_HA, _HB)]],
            g_v.at[pl.ds(_HA, _HB)], s_g2)

        d_z.wait()
        d_sh0 = pltpu.async_copy(
            zbuf, shared.at[pl.ds(sid * _APT, _ZB)], s_sh)
        d_sh1 = pltpu.async_copy(
            zbuf, shared.at[pl.ds(sid * _APT + _ZB, _ZB)], s_sh)

        def stepA(i, carry):
            e = i * 16
            c16 = col_v[pl.ds(e, 16)]
            g16 = g_v[pl.ds(e, 16)]
            idxA[pl.ds(e, 16)] = g16 * _PITCH + c16 + _COFF
            return carry

        def stepB(i, carry):
            e = i * 16
            c16 = col_v[pl.ds(_HA + e, 16)]
            g16 = g_v[pl.ds(_HA + e, 16)]
            idxB[pl.ds(e, 16)] = g16 * _PITCH + c16 + _COFF
            return carry

        with jax.named_scope("ph1_input"):
            d_on.wait()
            d_sh0.wait()
            d_sh1.wait()
        with jax.named_scope("ph2_zbar"):
            plsc.subcore_barrier()
        with jax.named_scope("p3a_s0issue"):
            d_s0 = pltpu.async_copy(ones_v, shared.at[row_v], s_sc,
                                    add=True)
        with jax.named_scope("p3b_gwait1"):
            d_col.wait()
            d_g1.wait()
        with jax.named_scope("p3c_loopA"):
            lax.fori_loop(0, _HA // 16, stepA, 0)
        with jax.named_scope("p3d_s1issue"):
            d_s1 = pltpu.async_copy(ones_v.at[pl.ds(0, _HA)],
                                    shared.at[idxA], s_sc, add=True)
        with jax.named_scope("p3e_gwait2"):
            d_g2.wait()
        with jax.named_scope("p3f_loopB"):
            lax.fori_loop(0, _HB // 16, stepB, 0)
        with jax.named_scope("p3g_s2issue"):
            d_s2 = pltpu.async_copy(ones_v.at[pl.ds(0, _HB)],
                                    shared.at[idxB], s_sc, add=True)
        with jax.named_scope("ph4_drain"):
            d_s0.wait()
            d_s1.wait()
            d_s2.wait()
        with jax.named_scope("ph5_sbar"):
            plsc.subcore_barrier()
        hb = _ZB // 2
        src0 = cid * _ACC + sid * _APT

        def rd(k, buf):
            return pltpu.async_copy(
                shared.at[pl.ds(sid * _APT + k * hb, hb)], buf, s_rd)

        def wr(k, buf):
            return pltpu.async_copy(
                buf, out_hbm.at[pl.ds(src0 + k * hb, hb)], s_wr)

        bufA = zbuf.at[pl.ds(0, hb)]
        bufB = zbuf.at[pl.ds(hb, hb)]
        r0 = rd(0, bufA)
        r1 = rd(1, bufB)
        r0.wait()
        w0 = wr(0, bufA)
        r1.wait()
        w1 = wr(1, bufB)
        w0.wait()
        r2 = rd(2, bufA)
        w1.wait()
        r3 = rd(3, bufB)
        r2.wait()
        w2 = wr(2, bufA)
        r3.wait()
        w3 = wr(3, bufB)
        with jax.named_scope("ph6_writeout"):
            w2.wait()
            w3.wait()

    return hist(edge_index, batch, ones, zeros)


def _tc_body(flat_ref, batch_ref, x_ref, wphi_ref, bphi_ref,
             w1_ref, b1_ref, w2_ref, b2_ref, out_ref, dv, c0, c1, sem):
    descs_d = [pltpu.make_async_copy(
        flat_ref.at[pl.ds(k * _ACC, _PITCH)], dv.at[k], sem)
        for k in range(_NC)]
    descs_c = []
    for g in range(_N_GRAPHS):
        descs_c.append(pltpu.make_async_copy(
            flat_ref.at[pl.ds(_COFF + g * _PITCH, _PITCH)],
            c0.at[g], sem))
        descs_c.append(pltpu.make_async_copy(
            flat_ref.at[pl.ds(_ACC + _COFF + g * _PITCH, _PITCH)],
            c1.at[g], sem))
    for d in descs_d:
        d.start()
    for d in descs_c:
        d.start()
    for d in descs_d:
        d.wait()
    deg = (dv[0:1, :] + dv[1:2, :])[:, :_N_NODES]
    gids = lax.broadcasted_iota(jnp.int32, (_N_GRAPHS, _N_NODES), 0)
    crow = jnp.where(gids == batch_ref[...], deg, 0.0)
    cnt = jnp.sum(crow, axis=1, keepdims=True)
    Yr = lax.dot_general(
        crow, x_ref[...], (((1,), (0,)), ((), ())),
        precision=lax.Precision.HIGHEST,
        preferred_element_type=jnp.float32)
    for d in descs_c:
        d.wait()
    ccol = (c0[...] + c1[...])[:, :_N_NODES]
    Yc = lax.dot_general(
        ccol, x_ref[...], (((1,), (0,)), ((), ())),
        precision=lax.Precision.HIGHEST,
        preferred_element_type=jnp.float32)
    pooled = (
        jnp.dot(Yr, wphi_ref[:_D, :], precision=lax.Precision.HIGHEST)
        + jnp.dot(Yc, wphi_ref[_D:, :], precision=lax.Precision.HIGHEST)
        + cnt * bphi_ref[...])
    h = jnp.maximum(
        jnp.dot(pooled, w1_ref[...], precision=lax.Precision.HIGHEST)
        + b1_ref[...], 0.0)
    out_ref[...] = (
        jnp.dot(h, w2_ref[...], precision=lax.Precision.HIGHEST)
        + b2_ref[...])


def kernel(x, edge_index, batch, W_phi, b_phi, W1, b1, W2, b2):
    ones = jnp.ones((_EPW,), jnp.float32)
    zeros = jnp.zeros((_ZB,), jnp.float32)

    flat = _sc_histogram(edge_index.reshape(2 * _N_EDGES), batch, ones,
                         zeros)

    out = pl.pallas_call(
        _tc_body,
        out_shape=jax.ShapeDtypeStruct((_N_GRAPHS, _N_CLASSES), jnp.float32),
        in_specs=[
            pl.BlockSpec(memory_space=pl.ANY),
            pl.BlockSpec(memory_space=pltpu.MemorySpace.VMEM),
            pl.BlockSpec(memory_space=pltpu.MemorySpace.VMEM),
            pl.BlockSpec(memory_space=pltpu.MemorySpace.VMEM),
            pl.BlockSpec(memory_space=pltpu.MemorySpace.VMEM),
            pl.BlockSpec(memory_space=pltpu.MemorySpace.VMEM),
            pl.BlockSpec(memory_space=pltpu.MemorySpace.VMEM),
            pl.BlockSpec(memory_space=pltpu.MemorySpace.VMEM),
            pl.BlockSpec(memory_space=pltpu.MemorySpace.VMEM),
        ],
        scratch_shapes=[
            pltpu.VMEM((_NC, _PITCH), jnp.float32),
            pltpu.VMEM((_N_GRAPHS, _PITCH), jnp.float32),
            pltpu.VMEM((_N_GRAPHS, _PITCH), jnp.float32),
            pltpu.SemaphoreType.DMA,
        ],
    )(flat, batch.reshape(1, _N_NODES), x, W_phi,
      b_phi.reshape(1, _D), W1, b1.reshape(1, _D), W2,
      b2.reshape(1, _N_CLASSES))
    return out


# speedup vs baseline: 1.5279x; 1.5279x over previous
import functools

import jax
import jax.numpy as jnp
from jax import lax
from jax.experimental import pallas as pl
from jax.experimental.pallas import tpu as pltpu
from jax.experimental.pallas import tpu_sc as plsc

_N_NODES = 10000
_N_EDGES = 320000
_D = 128
_N_GRAPHS = 64
_N_CLASSES = 10

_NC = 2
_NS = 16
_NW = _NC * _NS
_EPW = _N_EDGES // _NW
_PITCH = 10112
_COFF = _PITCH
_ACC = 657408
_APT = _ACC // _NS
_ZB = _APT // 2


_CHUNKS = _EPW // 16
_HA = 4992
_HB = _EPW - _HA


def _sc_histogram(edge_index, batch, ones, zeros):
    mesh = plsc.VectorSubcoreMesh(
        core_axis_name="c", subcore_axis_name="s",
        num_cores=_NC, num_subcores=_NS)

    @functools.partial(
        pl.kernel,
        out_type=jax.ShapeDtypeStruct((_NC * _ACC,), jnp.float32),
        mesh=mesh,
        scratch_types=[
            pltpu.VMEM((_EPW,), jnp.int32),
            pltpu.VMEM((_EPW,), jnp.int32),
            pltpu.VMEM((_EPW,), jnp.int32),
            pltpu.VMEM((_HA,), jnp.int32),
            pltpu.VMEM((_HB,), jnp.int32),
            pltpu.VMEM((_EPW,), jnp.float32),
            pltpu.VMEM((_ZB,), jnp.float32),
            pltpu.VMEM_SHARED((_ACC,), jnp.float32),
            pltpu.VMEM_SHARED((_N_NODES,), jnp.int32),
            pltpu.SemaphoreType.DMA,
            pltpu.SemaphoreType.DMA,
            pltpu.SemaphoreType.DMA,
            pltpu.SemaphoreType.DMA,
            pltpu.SemaphoreType.DMA,
            pltpu.SemaphoreType.DMA,
            pltpu.SemaphoreType.DMA,
            pltpu.SemaphoreType.DMA,
            pltpu.SemaphoreType.DMA,
            pltpu.SemaphoreType.DMA,
            pltpu.SemaphoreType.DMA,
        ],
    )
    def hist(edge_hbm, batch_hbm, ones_hbm, zeros_hbm, out_hbm,
             row_v, col_v, g_v, idxA, idxB, ones_v, zbuf, shared, batch_sh,
             s_b, s_row, s_col, s_ones, s_zero, s_sh, s_g1, s_g2, s_sc,
             s_rd, s_wr):
        cid = lax.axis_index("c")
        sid = lax.axis_index("s")
        wid = sid * _NC + cid
        base = wid * _EPW

        d_row = pltpu.async_copy(edge_hbm.at[pl.ds(base, _EPW)], row_v,
                                 s_row)
        d_col = pltpu.async_copy(edge_hbm.at[pl.ds(_N_EDGES + base, _EPW)],
                                 col_v, s_col)
        d_on = pltpu.async_copy(ones_hbm, ones_v, s_ones)
        d_z = pltpu.async_copy(zeros_hbm, zbuf, s_zero)

        @pl.when(sid == 0)
        def _():
            pltpu.sync_copy(batch_hbm, g_v)
            pltpu.sync_copy(g_v, batch_sh)

        plsc.subcore_barrier()
        d_row.wait()
        d_g1 = pltpu.async_copy(
            batch_sh.at[row_v.at[pl.ds(0, _HA)]],
            g_v.at[pl.ds(0, _HA)], s_g1)
        d_g2 = pltpu.async_copy(
            batch_sh.at[row_v.at[pl.ds(_HA, _HB)]],
            g_v.at[pl.ds(_HA, _HB)], s_g2)

        d_z.wait()
        d_sh0 = pltpu.async_copy(
            zbuf, shared.at[pl.ds(sid * _APT, _ZB)], s_sh)
        d_sh1 = pltpu.async_copy(
            zbuf, shared.at[pl.ds(sid * _APT + _ZB, _ZB)], s_sh)

        def stepA(i, carry):
            e = i * 16
            c16 = col_v[pl.ds(e, 16)]
            g16 = g_v[pl.ds(e, 16)]
            idxA[pl.ds(e, 16)] = g16 * _PITCH + c16 + _COFF
            return carry

        def stepB(i, carry):
            e = i * 16
            c16 = col_v[pl.ds(_HA + e, 16)]
            g16 = g_v[pl.ds(_HA + e, 16)]
            idxB[pl.ds(e, 16)] = g16 * _PITCH + c16 + _COFF
            return carry

        d_on.wait()
        d_sh0.wait()
        d_sh1.wait()
        plsc.subcore_barrier()
        d_s0 = pltpu.async_copy(ones_v, shared.at[row_v], s_sc, add=True)
        d_col.wait()
        d_g1.wait()
        lax.fori_loop(0, _HA // 16, stepA, 0)
        d_s1 = pltpu.async_copy(ones_v.at[pl.ds(0, _HA)], shared.at[idxA],
                                s_sc, add=True)
        d_g2.wait()
        lax.fori_loop(0, _HB // 16, stepB, 0)
        d_s2 = pltpu.async_copy(ones_v.at[pl.ds(0, _HB)], shared.at[idxB],
                                s_sc, add=True)
        d_s0.wait()
        d_s1.wait()
        d_s2.wait()
        plsc.subcore_barrier()
        hb = _ZB // 2
        src0 = cid * _ACC + sid * _APT

        def rd(k, buf):
            return pltpu.async_copy(
                shared.at[pl.ds(sid * _APT + k * hb, hb)], buf, s_rd)

        def wr(k, buf):
            return pltpu.async_copy(
                buf, out_hbm.at[pl.ds(src0 + k * hb, hb)], s_wr)

        bufA = zbuf.at[pl.ds(0, hb)]
        bufB = zbuf.at[pl.ds(hb, hb)]
        r0 = rd(0, bufA)
        r1 = rd(1, bufB)
        r0.wait()
        w0 = wr(0, bufA)
        r1.wait()
        w1 = wr(1, bufB)
        w0.wait()
        r2 = rd(2, bufA)
        w1.wait()
        r3 = rd(3, bufB)
        r2.wait()
        w2 = wr(2, bufA)
        r3.wait()
        w3 = wr(3, bufB)
        w2.wait()
        w3.wait()

    return hist(edge_index, batch, ones, zeros)


def _tc_body(flat_ref, batch_ref, x_ref, wphi_ref, bphi_ref,
             w1_ref, b1_ref, w2_ref, b2_ref, out_ref, dv, c0, c1, sem):
    descs_d = [pltpu.make_async_copy(
        flat_ref.at[pl.ds(k * _ACC, _PITCH)], dv.at[k], sem)
        for k in range(_NC)]
    descs_c = []
    for g in range(_N_GRAPHS):
        descs_c.append(pltpu.make_async_copy(
            flat_ref.at[pl.ds(_COFF + g * _PITCH, _PITCH)],
            c0.at[g], sem))
        descs_c.append(pltpu.make_async_copy(
            flat_ref.at[pl.ds(_ACC + _COFF + g * _PITCH, _PITCH)],
            c1.at[g], sem))
    for d in descs_d:
        d.start()
    for d in descs_c:
        d.start()
    for d in descs_d:
        d.wait()
    deg = (dv[0:1, :] + dv[1:2, :])[:, :_N_NODES]
    gids = lax.broadcasted_iota(jnp.int32, (_N_GRAPHS, _N_NODES), 0)
    crow = jnp.where(gids == batch_ref[...], deg, 0.0)
    cnt = jnp.sum(crow, axis=1, keepdims=True)
    Yr = lax.dot_general(
        crow, x_ref[...], (((1,), (0,)), ((), ())),
        precision=lax.Precision.HIGHEST,
        preferred_element_type=jnp.float32)
    for d in descs_c:
        d.wait()
    ccol = (c0[...] + c1[...])[:, :_N_NODES]
    Yc = lax.dot_general(
        ccol, x_ref[...], (((1,), (0,)), ((), ())),
        precision=lax.Precision.HIGHEST,
        preferred_element_type=jnp.float32)
    pooled = (
        jnp.dot(Yr, wphi_ref[:_D, :], precision=lax.Precision.HIGHEST)
        + jnp.dot(Yc, wphi_ref[_D:, :], precision=lax.Precision.HIGHEST)
        + cnt * bphi_ref[...])
    h = jnp.maximum(
        jnp.dot(pooled, w1_ref[...], precision=lax.Precision.HIGHEST)
        + b1_ref[...], 0.0)
    out_ref[...] = (
        jnp.dot(h, w2_ref[...], precision=lax.Precision.HIGHEST)
        + b2_ref[...])


def kernel(x, edge_index, batch, W_phi, b_phi, W1, b1, W2, b2):
    ones = jnp.ones((_EPW,), jnp.float32)
    zeros = jnp.zeros((_ZB,), jnp.float32)

    flat = _sc_histogram(edge_index.reshape(2 * _N_EDGES), batch, ones,
                         zeros)

    out = pl.pallas_call(
        _tc_body,
        out_shape=jax.ShapeDtypeStruct((_N_GRAPHS, _N_CLASSES), jnp.float32),
        in_specs=[
            pl.BlockSpec(memory_space=pl.ANY),
            pl.BlockSpec(memory_space=pltpu.MemorySpace.VMEM),
            pl.BlockSpec(memory_space=pltpu.MemorySpace.VMEM),
            pl.BlockSpec(memory_space=pltpu.MemorySpace.VMEM),
            pl.BlockSpec(memory_space=pltpu.MemorySpace.VMEM),
            pl.BlockSpec(memory_space=pltpu.MemorySpace.VMEM),
            pl.BlockSpec(memory_space=pltpu.MemorySpace.VMEM),
            pl.BlockSpec(memory_space=pltpu.MemorySpace.VMEM),
            pl.BlockSpec(memory_space=pltpu.MemorySpace.VMEM),
        ],
        scratch_shapes=[
            pltpu.VMEM((_NC, _PITCH), jnp.float32),
            pltpu.VMEM((_N_GRAPHS, _PITCH), jnp.float32),
            pltpu.VMEM((_N_GRAPHS, _PITCH), jnp.float32),
            pltpu.SemaphoreType.DMA,
        ],
    )(flat, batch.reshape(1, _N_NODES), x, W_phi,
      b_phi.reshape(1, _D), W1, b1.reshape(1, _D), W2,
      b2.reshape(1, _N_CLASSES))
    return out
